# baseline (speedup 1.0000x reference)
import functools

import jax
import jax.numpy as jnp
from jax import lax
from jax.experimental import pallas as pl
from jax.experimental.pallas import tpu as pltpu
from jax.experimental.pallas import tpu_sc as plsc

_B, _N, _DE = 8, 576, 384
_H, _K, _DIM, _DM = 2, 8192, 32, 32
_BN = _B * _N
_NW = 32
_TPW = _BN // _NW
_OUTW = _TPW * _DM


def _sc_gather_body(vals_hbm, idx_hbm, out_hbm, idxv, rows, outv, sem):
    c = lax.axis_index("c")
    s = lax.axis_index("s")
    w = s * 2 + c
    tb = w * _TPW
    pltpu.sync_copy(idx_hbm.at[pl.ds(tb, _TPW)], idxv.at[pl.ds(0, _TPW)])
    pltpu.sync_copy(idx_hbm.at[pl.ds(_BN + tb, _TPW)],
                    idxv.at[pl.ds(_TPW, _TPW)])
    cps = []
    for ch in range(4):
        cps.append(pltpu.async_copy(
            vals_hbm.at[idxv.at[pl.ds(ch * 72, 72)]],
            rows.at[pl.ds(ch * 72, 72)], sem))
    for cp in cps:
        cp.wait()

    def body(t, carry):
        for j in range(2):
            v0 = rows[t, pl.ds(j * 16, 16)]
            v1 = rows[_TPW + t, pl.ds(j * 16, 16)]
            outv[pl.ds(t * _DM + j * 16, 16)] = (v0 + v1) * 0.5
        return carry


    lax.fori_loop(0, _TPW, body, 0)
    pltpu.sync_copy(outv, out_hbm.at[pl.ds(w * _OUTW, _OUTW)])


@functools.partial(
    pl.kernel,
    out_type=jax.ShapeDtypeStruct((_BN * _DM,), jnp.float32),
    mesh=plsc.VectorSubcoreMesh(core_axis_name="c", subcore_axis_name="s"),
    scratch_types=[
        pltpu.VMEM((2 * _TPW,), jnp.int32),
        pltpu.VMEM((2 * _TPW, 128), jnp.float32),
        pltpu.VMEM((_OUTW,), jnp.float32),
        pltpu.SemaphoreType.DMA,
    ],
)
def _sc_gather(vals_hbm, idx_hbm, out_hbm, idxv, rows, outv, sem):
    _sc_gather_body(vals_hbm, idx_hbm, out_hbm, idxv, rows, outv, sem)


_NT = 128
_GRID = _BN // _NT
_KC = 512
_NKC = _K // _KC


def _tc_body(x_ref, rp_ref, cb_ref, out_ref):
    xt = x_ref[...]
    rp = rp_ref[...].astype(jnp.bfloat16)
    xp = lax.dot_general(xt, rp, (((1,), (0,)), ((), ())),
                         preferred_element_type=jnp.float32)
    lane = lax.broadcasted_iota(jnp.int32, (_NT, 128), 1)
    for h in range(_H):
        xph = xp[:, h * _DIM:(h + 1) * _DIM]
        xpb = xph.astype(jnp.bfloat16)
        cbh = cb_ref[h]
        a = jnp.sum(xph * xph, axis=1)
        ab = a[:, None]
        m_r = jnp.full((_NT, 128), jnp.inf, dtype=jnp.float32)
        i_r = jnp.zeros((_NT, 128), dtype=jnp.int32)
        for c in range(_NKC):
            cbc = cbh[c * _KC:(c + 1) * _KC, :]
            cc = jnp.sum(cbc * cbc, axis=1)
            s2 = lax.dot_general(xpb, cbc * 2.0,
                                 (((1,), (1,)), ((), ())),
                                 preferred_element_type=jnp.float32)
            d2 = (ab - s2) + cc[None, :]
            for sc in range(_KC // 128):
                d2s = d2[:, sc * 128:(sc + 1) * 128]
                lt = d2s < m_r
                m_r = jnp.where(lt, d2s, m_r)
                i_r = jnp.where(lt, c * (_KC // 128) + sc, i_r)
        m = jnp.min(m_r, axis=1)
        gi = i_r * 128 + lane
        idx = jnp.min(jnp.where(m_r == m[:, None], gi, 2 * _K), axis=1)
        out_ref[0, h] = idx + h * _K


def _tc_argmin(x2, rp2, cb):
    return pl.pallas_call(
        _tc_body,
        grid=(_GRID,),
        in_specs=[
            pl.BlockSpec((_NT, _DE), lambda i: (i, 0)),
            pl.BlockSpec((_DE, _H * _DIM), lambda i: (0, 0)),
            pl.BlockSpec((_H, _K, _DIM), lambda i: (0, 0, 0)),
        ],
        out_specs=pl.BlockSpec((1, _H, _NT), lambda i: (i, 0, 0)),
        out_shape=jax.ShapeDtypeStruct((_GRID, _H, _NT), jnp.int32),
    )(x2, rp2, cb)


def kernel(x, rand_proj, values, codebook):
    x2 = x.reshape(_BN, _DE)
    rp2 = rand_proj.transpose(1, 0, 2).reshape(_DE, _H * _DIM)
    idx = _tc_argmin(x2, rp2, codebook)
    idx_comb = idx.transpose(1, 0, 2).reshape(_H * _BN)
    vals_flat = jnp.pad(values.reshape(_H * _K, _DM),
                        ((0, 0), (0, 128 - _DM)))
    out = _sc_gather(vals_flat, idx_comb)
    return out.reshape(_B, _N, _DM)


# speedup vs baseline: 1.1604x; 1.1604x over previous
import functools

import jax
import jax.numpy as jnp
from jax import lax
from jax.experimental import pallas as pl
from jax.experimental.pallas import tpu as pltpu
from jax.experimental.pallas import tpu_sc as plsc

_B, _N, _DE = 8, 576, 384
_H, _K, _DIM, _DM = 2, 8192, 32, 32
_BN = _B * _N
_NW = 32
_TPW = _BN // _NW
_OUTW = _TPW * _DM


def _sc_gather_body(vals_hbm, idx_hbm, out_hbm, idxv, rows, outv, sem):
    c = lax.axis_index("c")
    s = lax.axis_index("s")
    w = s * 2 + c
    tb = w * _TPW
    pltpu.sync_copy(idx_hbm.at[pl.ds(tb, _TPW)], idxv.at[pl.ds(0, _TPW)])
    pltpu.sync_copy(idx_hbm.at[pl.ds(_BN + tb, _TPW)],
                    idxv.at[pl.ds(_TPW, _TPW)])
    cps = []
    for ch in range(4):
        cps.append(pltpu.async_copy(
            vals_hbm.at[idxv.at[pl.ds(ch * 72, 72)]],
            rows.at[pl.ds(ch * 72, 72)], sem))
    for cp in cps:
        cp.wait()

    def body(t, carry):
        for j in range(2):
            v0 = rows[t, pl.ds(j * 16, 16)]
            v1 = rows[_TPW + t, pl.ds(j * 16, 16)]
            outv[pl.ds(t * _DM + j * 16, 16)] = (v0 + v1) * 0.5
        return carry


    lax.fori_loop(0, _TPW, body, 0)
    pltpu.sync_copy(outv, out_hbm.at[pl.ds(w * _OUTW, _OUTW)])


@functools.partial(
    pl.kernel,
    out_type=jax.ShapeDtypeStruct((_BN * _DM,), jnp.float32),
    mesh=plsc.VectorSubcoreMesh(core_axis_name="c", subcore_axis_name="s"),
    scratch_types=[
        pltpu.VMEM((2 * _TPW,), jnp.int32),
        pltpu.VMEM((2 * _TPW, 128), jnp.float32),
        pltpu.VMEM((_OUTW,), jnp.float32),
        pltpu.SemaphoreType.DMA,
    ],
)
def _sc_gather(vals_hbm, idx_hbm, out_hbm, idxv, rows, outv, sem):
    _sc_gather_body(vals_hbm, idx_hbm, out_hbm, idxv, rows, outv, sem)


_NT = 256
_GRID = _BN // _NT
_KC = 512
_NKC = _K // _KC


def _tc_body(x_ref, rp_ref, cb_ref, out_ref):
    xt = x_ref[...]
    rp = rp_ref[...].astype(jnp.bfloat16)
    xp = lax.dot_general(xt, rp, (((1,), (0,)), ((), ())),
                         preferred_element_type=jnp.float32)
    lane = lax.broadcasted_iota(jnp.int32, (_NT, 128), 1)
    for h in range(_H):
        xph = xp[:, h * _DIM:(h + 1) * _DIM]
        xpb = xph.astype(jnp.bfloat16)
        cbh = cb_ref[h]
        a = jnp.sum(xph * xph, axis=1)
        ab = a[:, None]
        m_r = jnp.full((_NT, 128), jnp.inf, dtype=jnp.float32)
        i_r = jnp.zeros((_NT, 128), dtype=jnp.int32)
        for c in range(_NKC):
            cbc = cbh[c * _KC:(c + 1) * _KC, :]
            cc = jnp.sum(cbc * cbc, axis=1)
            s2 = lax.dot_general(xpb, cbc * 2.0,
                                 (((1,), (1,)), ((), ())),
                                 preferred_element_type=jnp.float32)
            d2 = (ab - s2) + cc[None, :]
            for sc in range(_KC // 128):
                d2s = d2[:, sc * 128:(sc + 1) * 128]
                lt = d2s < m_r
                m_r = jnp.where(lt, d2s, m_r)
                i_r = jnp.where(lt, c * (_KC // 128) + sc, i_r)
        m = jnp.min(m_r, axis=1)
        gi = i_r * 128 + lane
        idx = jnp.min(jnp.where(m_r == m[:, None], gi, 2 * _K), axis=1)
        out_ref[0, h] = idx + h * _K


def _tc_argmin(x2, rp2, cb):
    return pl.pallas_call(
        _tc_body,
        grid=(_GRID,),
        in_specs=[
            pl.BlockSpec((_NT, _DE), lambda i: (i, 0)),
            pl.BlockSpec((_DE, _H * _DIM), lambda i: (0, 0)),
            pl.BlockSpec((_H, _K, _DIM), lambda i: (0, 0, 0)),
        ],
        out_specs=pl.BlockSpec((1, _H, _NT), lambda i: (i, 0, 0)),
        out_shape=jax.ShapeDtypeStruct((_GRID, _H, _NT), jnp.int32),
    )(x2, rp2, cb)


def kernel(x, rand_proj, values, codebook):
    x2 = x.reshape(_BN, _DE)
    rp2 = rand_proj.transpose(1, 0, 2).reshape(_DE, _H * _DIM)
    idx = _tc_argmin(x2, rp2, codebook)
    idx_comb = idx.transpose(1, 0, 2).reshape(_H * _BN)
    vals_flat = jnp.pad(values.reshape(_H * _K, _DM),
                        ((0, 0), (0, 128 - _DM)))
    out = _sc_gather(vals_flat, idx_comb)
    return out.reshape(_B, _N, _DM)


# speedup vs baseline: 1.4975x; 1.2905x over previous
import functools

import jax
import jax.numpy as jnp
from jax import lax
from jax.experimental import pallas as pl
from jax.experimental.pallas import tpu as pltpu
from jax.experimental.pallas import tpu_sc as plsc

_B, _N, _DE = 8, 576, 384
_H, _K, _DIM, _DM = 2, 8192, 32, 32
_BN = _B * _N
_NW = 32
_TPW = _BN // _NW
_OUTW = _TPW * _DM


def _sc_gather_body(vals_hbm, idx_hbm, out_hbm, idxv, rows, outv, sem):
    c = lax.axis_index("c")
    s = lax.axis_index("s")
    w = s * 2 + c
    tb = w * _TPW
    pltpu.sync_copy(idx_hbm.at[pl.ds(tb, _TPW)], idxv.at[pl.ds(0, _TPW)])
    pltpu.sync_copy(idx_hbm.at[pl.ds(_BN + tb, _TPW)],
                    idxv.at[pl.ds(_TPW, _TPW)])
    cps = []
    for ch in range(4):
        cps.append(pltpu.async_copy(
            vals_hbm.at[idxv.at[pl.ds(ch * 72, 72)]],
            rows.at[pl.ds(ch * 72, 72)], sem))
    for cp in cps:
        cp.wait()

    def body(t, carry):
        for j in range(2):
            v0 = rows[t, pl.ds(j * 16, 16)]
            v1 = rows[_TPW + t, pl.ds(j * 16, 16)]
            outv[pl.ds(t * _DM + j * 16, 16)] = (v0 + v1) * 0.5
        return carry


    lax.fori_loop(0, _TPW, body, 0)
    pltpu.sync_copy(outv, out_hbm.at[pl.ds(w * _OUTW, _OUTW)])


@functools.partial(
    pl.kernel,
    out_type=jax.ShapeDtypeStruct((_BN * _DM,), jnp.float32),
    mesh=plsc.VectorSubcoreMesh(core_axis_name="c", subcore_axis_name="s"),
    scratch_types=[
        pltpu.VMEM((2 * _TPW,), jnp.int32),
        pltpu.VMEM((2 * _TPW, 128), jnp.float32),
        pltpu.VMEM((_OUTW,), jnp.float32),
        pltpu.SemaphoreType.DMA,
    ],
)
def _sc_gather(vals_hbm, idx_hbm, out_hbm, idxv, rows, outv, sem):
    _sc_gather_body(vals_hbm, idx_hbm, out_hbm, idxv, rows, outv, sem)


_NT = 768
_GRID = _BN // _NT
_KC = 512
_NKC = _K // _KC


def _tc_body(x_ref, rp_ref, cb_ref, out_ref):
    xt = x_ref[...]
    rp = rp_ref[...].astype(jnp.bfloat16)
    xp = lax.dot_general(xt, rp, (((1,), (0,)), ((), ())),
                         preferred_element_type=jnp.float32)
    lane = lax.broadcasted_iota(jnp.int32, (_NT, 128), 1)
    for h in range(_H):
        xph = xp[:, h * _DIM:(h + 1) * _DIM]
        xpb = xph.astype(jnp.bfloat16)
        cbh = cb_ref[h]
        a = jnp.sum(xph * xph, axis=1)
        ab = a[:, None]
        m_r = jnp.full((_NT, 128), jnp.inf, dtype=jnp.float32)
        i_r = jnp.zeros((_NT, 128), dtype=jnp.int32)
        for c in range(_NKC):
            cbc = cbh[c * _KC:(c + 1) * _KC, :]
            cc = jnp.sum(cbc * cbc, axis=1)
            s2 = lax.dot_general(xpb, cbc * 2.0,
                                 (((1,), (1,)), ((), ())),
                                 preferred_element_type=jnp.float32)
            d2 = (ab - s2) + cc[None, :]
            for sc in range(_KC // 128):
                d2s = d2[:, sc * 128:(sc + 1) * 128]
                lt = d2s < m_r
                m_r = jnp.where(lt, d2s, m_r)
                i_r = jnp.where(lt, c * (_KC // 128) + sc, i_r)
        m = jnp.min(m_r, axis=1)
        gi = i_r * 128 + lane
        idx = jnp.min(jnp.where(m_r == m[:, None], gi, 2 * _K), axis=1)
        out_ref[0, h] = idx + h * _K


def _tc_argmin(x2, rp2, cb):
    return pl.pallas_call(
        _tc_body,
        grid=(_GRID,),
        in_specs=[
            pl.BlockSpec((_NT, _DE), lambda i: (i, 0)),
            pl.BlockSpec((_DE, _H * _DIM), lambda i: (0, 0)),
            pl.BlockSpec((_H, _K, _DIM), lambda i: (0, 0, 0)),
        ],
        out_specs=pl.BlockSpec((1, _H, _NT), lambda i: (i, 0, 0)),
        out_shape=jax.ShapeDtypeStruct((_GRID, _H, _NT), jnp.int32),
    )(x2, rp2, cb)


def kernel(x, rand_proj, values, codebook):
    x2 = x.reshape(_BN, _DE)
    rp2 = rand_proj.transpose(1, 0, 2).reshape(_DE, _H * _DIM)
    idx = _tc_argmin(x2, rp2, codebook)
    idx_comb = idx.transpose(1, 0, 2).reshape(_H * _BN)
    vals_flat = jnp.pad(values.reshape(_H * _K, _DM),
                        ((0, 0), (0, 128 - _DM)))
    out = _sc_gather(vals_flat, idx_comb)
    return out.reshape(_B, _N, _DM)


# speedup vs baseline: 1.5361x; 1.0258x over previous
import functools

import jax
import jax.numpy as jnp
from jax import lax
from jax.experimental import pallas as pl
from jax.experimental.pallas import tpu as pltpu
from jax.experimental.pallas import tpu_sc as plsc

_B, _N, _DE = 8, 576, 384
_H, _K, _DIM, _DM = 2, 8192, 32, 32
_BN = _B * _N
_NW = 32
_TPW = _BN // _NW
_OUTW = _TPW * _DM


def _sc_gather_body(vals_hbm, idx_hbm, out_hbm, idxv, rows, outv, sem):
    c = lax.axis_index("c")
    s = lax.axis_index("s")
    w = s * 2 + c
    tb = w * _TPW
    pltpu.sync_copy(idx_hbm.at[pl.ds(tb, _TPW)], idxv.at[pl.ds(0, _TPW)])
    pltpu.sync_copy(idx_hbm.at[pl.ds(_BN + tb, _TPW)],
                    idxv.at[pl.ds(_TPW, _TPW)])
    cps = []
    for ch in range(4):
        cps.append(pltpu.async_copy(
            vals_hbm.at[idxv.at[pl.ds(ch * 72, 72)]],
            rows.at[pl.ds(ch * 72, 72)], sem))
    for cp in cps:
        cp.wait()

    def body(t, carry):
        for j in range(2):
            v0 = rows[t, pl.ds(j * 16, 16)]
            v1 = rows[_TPW + t, pl.ds(j * 16, 16)]
            outv[pl.ds(t * _DM + j * 16, 16)] = (v0 + v1) * 0.5
        return carry


    lax.fori_loop(0, _TPW, body, 0)
    pltpu.sync_copy(outv, out_hbm.at[pl.ds(w * _OUTW, _OUTW)])


@functools.partial(
    pl.kernel,
    out_type=jax.ShapeDtypeStruct((_BN * _DM,), jnp.float32),
    mesh=plsc.VectorSubcoreMesh(core_axis_name="c", subcore_axis_name="s"),
    scratch_types=[
        pltpu.VMEM((2 * _TPW,), jnp.int32),
        pltpu.VMEM((2 * _TPW, 128), jnp.float32),
        pltpu.VMEM((_OUTW,), jnp.float32),
        pltpu.SemaphoreType.DMA,
    ],
)
def _sc_gather(vals_hbm, idx_hbm, out_hbm, idxv, rows, outv, sem):
    _sc_gather_body(vals_hbm, idx_hbm, out_hbm, idxv, rows, outv, sem)


_NT = 512
_GRID = _BN // _NT
_KC = 2048
_NKC = _K // _KC


def _tc_body(x_ref, rp_ref, cb_ref, out_ref):
    xt = x_ref[...]
    rp = rp_ref[...].astype(jnp.bfloat16)
    xp = lax.dot_general(xt, rp, (((1,), (0,)), ((), ())),
                         preferred_element_type=jnp.float32)
    lane = lax.broadcasted_iota(jnp.int32, (_NT, 128), 1)
    for h in range(_H):
        xph = xp[:, h * _DIM:(h + 1) * _DIM]
        xpb = xph.astype(jnp.bfloat16)
        cbh = cb_ref[h]
        a = jnp.sum(xph * xph, axis=1)
        ab = a[:, None]
        m_r = jnp.full((_NT, 128), jnp.inf, dtype=jnp.float32)
        i_r = jnp.zeros((_NT, 128), dtype=jnp.int32)
        for c in range(_NKC):
            cbc = cbh[c * _KC:(c + 1) * _KC, :]
            cc = jnp.sum(cbc * cbc, axis=1)
            s2 = lax.dot_general(xpb, cbc * 2.0,
                                 (((1,), (1,)), ((), ())),
                                 preferred_element_type=jnp.float32)
            d2 = (ab - s2) + cc[None, :]
            for sc in range(_KC // 128):
                d2s = d2[:, sc * 128:(sc + 1) * 128]
                lt = d2s < m_r
                m_r = jnp.where(lt, d2s, m_r)
                i_r = jnp.where(lt, c * (_KC // 128) + sc, i_r)
        m = jnp.min(m_r, axis=1)
        gi = i_r * 128 + lane
        idx = jnp.min(jnp.where(m_r == m[:, None], gi, 2 * _K), axis=1)
        out_ref[0, h] = idx + h * _K


def _tc_argmin(x2, rp2, cb):
    return pl.pallas_call(
        _tc_body,
        grid=(_GRID,),
        in_specs=[
            pl.BlockSpec((_NT, _DE), lambda i: (i, 0)),
            pl.BlockSpec((_DE, _H * _DIM), lambda i: (0, 0)),
            pl.BlockSpec((_H, _K, _DIM), lambda i: (0, 0, 0)),
        ],
        out_specs=pl.BlockSpec((1, _H, _NT), lambda i: (i, 0, 0)),
        out_shape=jax.ShapeDtypeStruct((_GRID, _H, _NT), jnp.int32),
    )(x2, rp2, cb)


def kernel(x, rand_proj, values, codebook):
    x2 = x.reshape(_BN, _DE)
    rp2 = rand_proj.transpose(1, 0, 2).reshape(_DE, _H * _DIM)
    idx = _tc_argmin(x2, rp2, codebook)
    idx_comb = idx.transpose(1, 0, 2).reshape(_H * _BN)
    vals_flat = jnp.pad(values.reshape(_H * _K, _DM),
                        ((0, 0), (0, 128 - _DM)))
    out = _sc_gather(vals_flat, idx_comb)
    return out.reshape(_B, _N, _DM)


# speedup vs baseline: 1.5937x; 1.0375x over previous
import functools

import jax
import jax.numpy as jnp
from jax import lax
from jax.experimental import pallas as pl
from jax.experimental.pallas import tpu as pltpu
from jax.experimental.pallas import tpu_sc as plsc

_B, _N, _DE = 8, 576, 384
_H, _K, _DIM, _DM = 2, 8192, 32, 32
_BN = _B * _N
_NW = 32
_TPW = _BN // _NW
_OUTW = _TPW * _DM


def _sc_gather_body(vals_hbm, idx_hbm, out_hbm, idxv, rows, outv, sem):
    c = lax.axis_index("c")
    s = lax.axis_index("s")
    w = s * 2 + c
    tb = w * _TPW
    pltpu.sync_copy(idx_hbm.at[pl.ds(tb, _TPW)], idxv.at[pl.ds(0, _TPW)])
    pltpu.sync_copy(idx_hbm.at[pl.ds(_BN + tb, _TPW)],
                    idxv.at[pl.ds(_TPW, _TPW)])
    cps = []
    for ch in range(4):
        cps.append(pltpu.async_copy(
            vals_hbm.at[idxv.at[pl.ds(ch * 72, 72)]],
            rows.at[pl.ds(ch * 72, 72)], sem))
    for cp in cps:
        cp.wait()

    def body(t, carry):
        for j in range(2):
            v0 = rows[t, pl.ds(j * 16, 16)]
            v1 = rows[_TPW + t, pl.ds(j * 16, 16)]
            outv[pl.ds(t * _DM + j * 16, 16)] = (v0 + v1) * 0.5
        return carry


    lax.fori_loop(0, _TPW, body, 0)
    pltpu.sync_copy(outv, out_hbm.at[pl.ds(w * _OUTW, _OUTW)])


@functools.partial(
    pl.kernel,
    out_type=jax.ShapeDtypeStruct((_BN * _DM,), jnp.float32),
    mesh=plsc.VectorSubcoreMesh(core_axis_name="c", subcore_axis_name="s"),
    scratch_types=[
        pltpu.VMEM((2 * _TPW,), jnp.int32),
        pltpu.VMEM((2 * _TPW, 128), jnp.float32),
        pltpu.VMEM((_OUTW,), jnp.float32),
        pltpu.SemaphoreType.DMA,
    ],
)
def _sc_gather(vals_hbm, idx_hbm, out_hbm, idxv, rows, outv, sem):
    _sc_gather_body(vals_hbm, idx_hbm, out_hbm, idxv, rows, outv, sem)


_NT = 512
_GRID = _BN // _NT
_KC = 2048
_NKC = _K // _KC


def _tc_body(x_ref, rp_ref, cb_ref, out_ref, cb2_ref, cc_ref):
    @pl.when(pl.program_id(0) == 0)
    def _init():
        for h in range(_H):
            cbh = cb_ref[h]
            cb2_ref[h] = cbh * 2.0
            cc_ref[h] = jnp.sum(cbh * cbh, axis=1)

    xt = x_ref[...]
    rp = rp_ref[...].astype(jnp.bfloat16)
    xp = lax.dot_general(xt, rp, (((1,), (0,)), ((), ())),
                         preferred_element_type=jnp.float32)
    lane = lax.broadcasted_iota(jnp.int32, (_NT, 128), 1)
    for h in range(_H):
        xph = xp[:, h * _DIM:(h + 1) * _DIM]
        xpb = xph.astype(jnp.bfloat16)
        a = jnp.sum(xph * xph, axis=1)
        ab = a[:, None]
        m_r = jnp.full((_NT, 128), jnp.inf, dtype=jnp.float32)
        i_r = jnp.zeros((_NT, 128), dtype=jnp.int32)
        for c in range(_NKC):
            cc = cc_ref[h, c * _KC:(c + 1) * _KC]
            s2 = lax.dot_general(xpb,
                                 cb2_ref[h, c * _KC:(c + 1) * _KC, :],
                                 (((1,), (1,)), ((), ())),
                                 preferred_element_type=jnp.float32)
            d2 = (ab - s2) + cc[None, :]
            for sc in range(_KC // 128):
                d2s = d2[:, sc * 128:(sc + 1) * 128]
                lt = d2s < m_r
                m_r = jnp.where(lt, d2s, m_r)
                i_r = jnp.where(lt, c * (_KC // 128) + sc, i_r)
        m = jnp.min(m_r, axis=1)
        gi = i_r * 128 + lane
        idx = jnp.min(jnp.where(m_r == m[:, None], gi, 2 * _K), axis=1)
        out_ref[0, h] = idx + h * _K


def _tc_argmin(x2, rp2, cb):
    return pl.pallas_call(
        _tc_body,
        grid=(_GRID,),
        in_specs=[
            pl.BlockSpec((_NT, _DE), lambda i: (i, 0)),
            pl.BlockSpec((_DE, _H * _DIM), lambda i: (0, 0)),
            pl.BlockSpec((_H, _K, _DIM), lambda i: (0, 0, 0)),
        ],
        out_specs=pl.BlockSpec((1, _H, _NT), lambda i: (i, 0, 0)),
        out_shape=jax.ShapeDtypeStruct((_GRID, _H, _NT), jnp.int32),
        scratch_shapes=[
            pltpu.VMEM((_H, _K, _DIM), jnp.float32),
            pltpu.VMEM((_H, _K), jnp.float32),
        ],
    )(x2, rp2, cb)


def kernel(x, rand_proj, values, codebook):
    x2 = x.reshape(_BN, _DE)
    rp2 = rand_proj.transpose(1, 0, 2).reshape(_DE, _H * _DIM)
    idx = _tc_argmin(x2, rp2, codebook)
    idx_comb = idx.transpose(1, 0, 2).reshape(_H * _BN)
    vals_flat = jnp.pad(values.reshape(_H * _K, _DM),
                        ((0, 0), (0, 128 - _DM)))
    out = _sc_gather(vals_flat, idx_comb)
    return out.reshape(_B, _N, _DM)
